# baseline (speedup 1.0000x reference)
import jax
import jax.numpy as jnp
import numpy as np
from jax import lax
from jax.experimental import pallas as pl
from jax.experimental.pallas import tpu as pltpu
from jax.experimental.pallas import tpu_sc as plsc

B, N = 2, 2048
D_IN, D = 128, 256
H, DH = 4, 64
GX, GY, GT = 128, 128, 200
TOT = GX * GY * GT
NO = 32
PT = N // 16
GSLC = TOT // 16

_OFF_LIST = [(dx, dy, dt) for dx in (-1, 0, 1) for dy in (-1, 0, 1) for dt in (-1, 0, 1)]
_OFFI = np.zeros((NO, 4), np.int32)
for _j, (_a, _b, _c) in enumerate(_OFF_LIST):
    _OFFI[_j] = (_a, _b, _c, 1)
_OFFS32 = np.zeros((NO, 3), np.float32)
for _j, (_a, _b, _c) in enumerate(_OFF_LIST):
    _OFFS32[_j] = (_a, _b, _c)
_SEL = np.zeros((D, H), np.float32)
for _h in range(H):
    _SEL[_h * DH:(_h + 1) * DH, _h] = 1.0


def _a_body(feat_ref, coordsT_ref, offi_ref, Wf_ref, bf_ref, Wq_ref, bq_ref,
            fp_ref, q_ref, nlinT_ref):
    f = feat_ref[0]
    fp = jnp.dot(f, Wf_ref[...], preferred_element_type=jnp.float32, precision=lax.Precision.HIGHEST) + bf_ref[...]
    fp_ref[0] = fp
    q_ref[0] = jnp.dot(fp, Wq_ref[...], preferred_element_type=jnp.float32, precision=lax.Precision.HIGHEST) + bq_ref[...]
    ct = coordsT_ref[0]
    vx = (jnp.clip(ct[0:1, :] * (1.0 / 256.0), 0.0, 1.0) * float(GX - 1)).astype(jnp.int32)
    vy = (jnp.clip(ct[1:2, :] * (1.0 / 256.0), 0.0, 1.0) * float(GY - 1)).astype(jnp.int32)
    vt = (jnp.clip(ct[2:3, :], 0.0, 1.0) * float(GT - 1)).astype(jnp.int32)
    nx = offi_ref[:, 0:1] + vx
    ny = offi_ref[:, 1:2] + vy
    nt = offi_ref[:, 2:3] + vt
    inb = ((nx >= 0) & (nx < GX) & (ny >= 0) & (ny < GY)
           & (nt >= 0) & (nt < GT) & (offi_ref[:, 3:4] > 0))
    nlin = nx * (GY * GT) + ny * GT + nt
    nlinT_ref[0] = jnp.where(inb, nlin, -1)


def _stage_a(features, coordsT, offi, W_feat, b_feat, Wq, bq):
    return pl.pallas_call(
        _a_body,
        grid=(B,),
        in_specs=[
            pl.BlockSpec((1, N, D_IN), lambda b: (b, 0, 0)),
            pl.BlockSpec((1, 3, N), lambda b: (b, 0, 0)),
            pl.BlockSpec((NO, 4), lambda b: (0, 0)),
            pl.BlockSpec((D_IN, D), lambda b: (0, 0)),
            pl.BlockSpec((1, D), lambda b: (0, 0)),
            pl.BlockSpec((D, D), lambda b: (0, 0)),
            pl.BlockSpec((1, D), lambda b: (0, 0)),
        ],
        out_specs=[
            pl.BlockSpec((1, N, D), lambda b: (b, 0, 0)),
            pl.BlockSpec((1, N, D), lambda b: (b, 0, 0)),
            pl.BlockSpec((1, NO, N), lambda b: (b, 0, 0)),
        ],
        out_shape=[
            jax.ShapeDtypeStruct((B, N, D), jnp.float32),
            jax.ShapeDtypeStruct((B, N, D), jnp.float32),
            jax.ShapeDtypeStruct((B, NO, N), jnp.int32),
        ],
    )(features, coordsT, offi, W_feat, b_feat.reshape(1, D), Wq, bq.reshape(1, D))


def _b1_body(nlinT_hbm, grid_hbm, fill_v, lin_v, val_v):
    c = lax.axis_index("c")
    t = lax.axis_index("s")
    i16 = lax.iota(jnp.int32, 16)
    gbase = c * TOT

    def _fill(i, _):
        fill_v[pl.ds(i * 16, 16)] = jnp.full((16,), -1, jnp.int32)
        return 0
    lax.fori_loop(0, 800, _fill, 0)

    def _small(g, _):
        val_v[pl.ds(g * 16, 16)] = t * PT + g * 16 + i16
        return 0
    lax.fori_loop(0, 8, _small, 0)

    def _memset(i, _):
        pltpu.sync_copy(fill_v, grid_hbm.at[pl.ds(gbase + t * GSLC + i * 12800, 12800)])
        return 0
    lax.fori_loop(0, 16, _memset, 0)

    pltpu.sync_copy(nlinT_hbm.at[pl.ds((c * NO + 13) * N + t * PT, PT)], lin_v)

    def _lclamp(g, _):
        lin_v[pl.ds(g * 16, 16)] = lin_v[pl.ds(g * 16, 16)] + gbase
        return 0
    lax.fori_loop(0, PT // 16, _lclamp, 0)

    plsc.subcore_barrier()

    pltpu.sync_copy(val_v, grid_hbm.at[lin_v])


def _b2_body(nlinT_hbm, grid_hbm, seg_hbm, repT_hbm,
             idxc_v, lin_v, seg_v, repc_v):
    c = lax.axis_index("c")
    t = lax.axis_index("s")
    gbase = c * TOT

    def _ldrow(j, _):
        pltpu.sync_copy(nlinT_hbm.at[pl.ds((c * NO + j) * N + t * PT, PT)], repc_v.at[j])
        return 0
    lax.fori_loop(0, NO, _ldrow, 0)
    pltpu.sync_copy(nlinT_hbm.at[pl.ds((c * NO + 13) * N + t * PT, PT)], lin_v)

    def _lclamp(g, _):
        lin_v[pl.ds(g * 16, 16)] = lin_v[pl.ds(g * 16, 16)] + gbase
        return 0
    lax.fori_loop(0, PT // 16, _lclamp, 0)

    def _clamp(i, _):
        idxc_v[i // 8, pl.ds((i % 8) * 16, 16)] = gbase + jnp.maximum(
            repc_v[i // 8, pl.ds((i % 8) * 16, 16)], 0)
        return 0
    lax.fori_loop(0, NO * PT // 16, _clamp, 0)

    pltpu.sync_copy(grid_hbm.at[lin_v], seg_v)
    pltpu.sync_copy(seg_v, seg_hbm.at[pl.ds(c * N + t * PT, PT)])

    def _ngather(j, _):
        pltpu.sync_copy(grid_hbm.at[idxc_v.at[j]], repc_v.at[j])
        return 0
    lax.fori_loop(0, NO, _ngather, 0)

    def _strow(j, _):
        pltpu.sync_copy(repc_v.at[j], repT_hbm.at[pl.ds((c * NO + j) * N + t * PT, PT)])
        return 0
    lax.fori_loop(0, NO, _strow, 0)


def _stage_b(nlinT_f):
    sc_mesh = plsc.VectorSubcoreMesh(core_axis_name="c", subcore_axis_name="s")
    b1 = pl.kernel(
        _b1_body,
        out_type=[jax.ShapeDtypeStruct((B * TOT,), jnp.int32)],
        mesh=sc_mesh,
        scratch_types=[
            pltpu.VMEM((12800,), jnp.int32),
            pltpu.VMEM((PT,), jnp.int32),
            pltpu.VMEM((PT,), jnp.int32),
        ],
    )
    grid = b1(nlinT_f)[0]
    b2 = pl.kernel(
        _b2_body,
        out_type=[
            jax.ShapeDtypeStruct((B * N,), jnp.int32),
            jax.ShapeDtypeStruct((B * NO * N,), jnp.int32),
        ],
        mesh=plsc.VectorSubcoreMesh(core_axis_name="c", subcore_axis_name="s"),
        scratch_types=[
            pltpu.VMEM((NO, PT), jnp.int32),
            pltpu.VMEM((PT,), jnp.int32),
            pltpu.VMEM((PT,), jnp.int32),
            pltpu.VMEM((NO, PT), jnp.int32),
        ],
    )
    seg_f, repT_f = b2(nlinT_f, grid)
    return grid, seg_f, repT_f


_SN = 256


def _c_body(fp_ref, seg_ref, offs_ref, Wia_ref, bia_ref, Wp1_ref, bp1_ref,
            Wp2_ref, bp2_ref, Wk_ref, bk_ref, Wv_ref, bv_ref,
            kvp_ref, pe_ref):
    i = pl.program_id(1)
    seg = seg_ref[0]
    uio = lax.broadcasted_iota(jnp.int32, (_SN, 1), 0) + i * _SN
    pmat = (uio == seg).astype(jnp.float32)
    cnt = jnp.maximum(jnp.sum(pmat, axis=1, keepdims=True), 1.0)
    sums = jnp.dot(pmat, fp_ref[0], preferred_element_type=jnp.float32, precision=lax.Precision.HIGHEST)
    means = sums * (1.0 / cnt)
    vf = jnp.maximum(
        jnp.dot(means, Wia_ref[...], preferred_element_type=jnp.float32, precision=lax.Precision.HIGHEST) + bia_ref[...], 0.0)
    kvf = jnp.dot(vf, Wk_ref[...], preferred_element_type=jnp.float32, precision=lax.Precision.HIGHEST)
    vvf = jnp.dot(vf, Wv_ref[...], preferred_element_type=jnp.float32, precision=lax.Precision.HIGHEST)

    def _pack(x):
        lo = lax.bitcast_convert_type(
            x[:, 0:D // 2].astype(jnp.bfloat16).astype(jnp.float32), jnp.int32)
        hi = lax.bitcast_convert_type(
            x[:, D // 2:D].astype(jnp.bfloat16).astype(jnp.float32), jnp.int32)
        return hi | ((lo >> 16) & 0xffff)
    kvp_ref[0] = jnp.concatenate([_pack(kvf), _pack(vvf)], axis=1)
    offs = offs_ref[...]
    pe_h = jnp.maximum(jnp.dot(offs, Wp1_ref[...], preferred_element_type=jnp.float32, precision=lax.Precision.HIGHEST)
                       + bp1_ref[...], 0.0)
    pe = jnp.dot(pe_h, Wp2_ref[...], preferred_element_type=jnp.float32, precision=lax.Precision.HIGHEST) + bp2_ref[...]
    kpe = jnp.dot(pe, Wk_ref[...], preferred_element_type=jnp.float32, precision=lax.Precision.HIGHEST) + bk_ref[...]
    vpe = jnp.dot(pe, Wv_ref[...], preferred_element_type=jnp.float32, precision=lax.Precision.HIGHEST) + bv_ref[...]
    pe_ref[...] = jnp.concatenate([kpe, vpe], axis=0)


def _stage_c(fp, seg2, offsf, W_ia, b_ia, W_p1, b_p1, W_p2, b_p2, Wk, bk, Wv, bv):
    return pl.pallas_call(
        _c_body,
        grid=(B, N // _SN),
        in_specs=[
            pl.BlockSpec((1, N, D), lambda b, i: (b, 0, 0)),
            pl.BlockSpec((1, 1, N), lambda b, i: (b, 0, 0)),
            pl.BlockSpec((NO, 3), lambda b, i: (0, 0)),
            pl.BlockSpec((D, D), lambda b, i: (0, 0)),
            pl.BlockSpec((1, D), lambda b, i: (0, 0)),
            pl.BlockSpec((3, D // 2), lambda b, i: (0, 0)),
            pl.BlockSpec((1, D // 2), lambda b, i: (0, 0)),
            pl.BlockSpec((D // 2, D), lambda b, i: (0, 0)),
            pl.BlockSpec((1, D), lambda b, i: (0, 0)),
            pl.BlockSpec((D, D), lambda b, i: (0, 0)),
            pl.BlockSpec((1, D), lambda b, i: (0, 0)),
            pl.BlockSpec((D, D), lambda b, i: (0, 0)),
            pl.BlockSpec((1, D), lambda b, i: (0, 0)),
        ],
        out_specs=[
            pl.BlockSpec((1, _SN, D), lambda b, i: (b, i, 0)),
            pl.BlockSpec((2 * NO, D), lambda b, i: (0, 0)),
        ],
        out_shape=[
            jax.ShapeDtypeStruct((B, N, D), jnp.int32),
            jax.ShapeDtypeStruct((2 * NO, D), jnp.float32),
        ],
    )(fp, seg2, offsf, W_ia, b_ia.reshape(1, D), W_p1, b_p1.reshape(1, D // 2),
      W_p2, b_p2.reshape(1, D), Wk, bk.reshape(1, D), Wv, bv.reshape(1, D))


_DC = PT
_NCH = 27


def _d_body(kvp_hbm, repT_hbm, kg_hbm,
            raw_v, repc_v, kb0, kb1,
            kg0, kg1, kg2, kg3, kw0, kw1):
    c = lax.axis_index("c")
    t = lax.axis_index("s")
    kbufs = (kb0, kb1)
    kgs = ((kg0, kg2), (kg1, kg3))
    kws = (kw0, kw1)

    def _ldrow(j, _):
        pltpu.sync_copy(repT_hbm.at[pl.ds((c * NO + j) * N + t * PT, PT)], raw_v.at[j])
        return 0
    lax.fori_loop(0, _NCH, _ldrow, 0)

    def _clamp(i, _):
        repc_v[pl.ds(i * 16, 16)] = c * N + jnp.maximum(raw_v[i // 8, pl.ds((i % 8) * 16, 16)], 0)
        return 0
    lax.fori_loop(0, _NCH * PT // 16, _clamp, 0)

    _HD = _DC // 2

    def _startg(i, p):
        for s in range(2):
            idx = repc_v.at[pl.ds(i * _DC + s * _HD, _HD)]
            pltpu.async_copy(kvp_hbm.at[idx], kbufs[p].at[pl.ds(s * _HD, _HD), :], kgs[p][s])

    def _waitg(i, p):
        for s in range(2):
            idx = repc_v.at[pl.ds(i * _DC + s * _HD, _HD)]
            pltpu.make_async_copy(kvp_hbm.at[idx], kbufs[p].at[pl.ds(s * _HD, _HD), :], kgs[p][s]).wait()

    def _out(i):
        return (c * NO + i) * N + t * PT

    def _drainw(p):
        pltpu.make_async_copy(kbufs[p], kg_hbm.at[pl.ds(0, _DC), :], kws[p]).wait()

    _startg(0, 0)

    def _pair(g, _):
        for p in (0, 1):
            i = 2 * g + p
            q = 1 - p
            if p == 0:
                @pl.when(g >= 1)
                def _():
                    _drainw(q)
                _startg(i + 1, q)
            else:
                _drainw(q)
                _startg(i + 1, q)
            _waitg(i, p)
            pltpu.async_copy(kbufs[p], kg_hbm.at[pl.ds(_out(i), _DC), :], kws[p])
        return 0
    lax.fori_loop(0, (_NCH - 1) // 2, _pair, 0)

    _waitg(_NCH - 1, 0)
    pltpu.async_copy(kbufs[0], kg_hbm.at[pl.ds(_out(_NCH - 1), _DC), :], kws[0])
    _drainw(0)
    _drainw(1)


def _stage_d(kvp_f, repT_f):
    f = pl.kernel(
        _d_body,
        out_type=[
            jax.ShapeDtypeStruct((B * NO * N, D), jnp.int32),
        ],
        mesh=plsc.VectorSubcoreMesh(core_axis_name="c", subcore_axis_name="s"),
        scratch_types=[
            pltpu.VMEM((NO, PT), jnp.int32),
            pltpu.VMEM((NO * PT,), jnp.int32),
            pltpu.VMEM((_DC, D), jnp.int32),
            pltpu.VMEM((_DC, D), jnp.int32),
            pltpu.SemaphoreType.DMA,
            pltpu.SemaphoreType.DMA,
            pltpu.SemaphoreType.DMA,
            pltpu.SemaphoreType.DMA,
            pltpu.SemaphoreType.DMA,
            pltpu.SemaphoreType.DMA,
        ],
    )
    return f(kvp_f, repT_f)[0]


_BN = 128


def _e_body(q_ref, fp_ref, nlinT_ref, repT_ref, kg_ref, pe_ref,
            sel_ref, selT_ref, Wo_ref, bo_ref, g_ref, be_ref, out_ref):
    q = q_ref[0]
    found = (nlinT_ref[0] >= 0) & (repT_ref[0] >= 0)
    pen = jnp.where(found, 0.0, -1e9)
    kpe = pe_ref[0:NO, :]
    vpe = pe_ref[NO:2 * NO, :]
    Dh = D // 2
    validm = (lax.broadcasted_iota(jnp.int32, (NO, 1), 0) < 27).astype(jnp.int32)
    kvw = kg_ref[0] * validm[:, :, None]
    kw = kvw[:, :, 0:Dh]
    vw = kvw[:, :, Dh:D]

    def _lo(w):
        return lax.bitcast_convert_type(w << 16, jnp.float32)

    def _hi(w):
        return lax.bitcast_convert_type(w & jnp.int32(-65536), jnp.float32)

    klo = _lo(kw) + kpe[:, None, 0:Dh]
    khi = _hi(kw) + kpe[:, None, Dh:D]
    plo = q[None, :, 0:Dh] * klo
    phi = q[None, :, Dh:D] * khi
    logits = (jnp.dot(plo.reshape(NO * _BN, Dh), sel_ref[0:Dh, :],
                      preferred_element_type=jnp.float32, precision=lax.Precision.HIGHEST)
              + jnp.dot(phi.reshape(NO * _BN, Dh), sel_ref[Dh:D, :],
                        preferred_element_type=jnp.float32, precision=lax.Precision.HIGHEST)
              ).reshape(NO, _BN, H)
    logits = logits * (1.0 / np.sqrt(DH).astype(np.float32)) + pen[:, :, None]
    m = jnp.max(logits, axis=0, keepdims=True)
    p = jnp.exp(logits - m)
    attn = p / jnp.sum(p, axis=0, keepdims=True)
    af = attn.reshape(NO * _BN, H)
    alo = jnp.dot(af, selT_ref[:, 0:Dh],
                  preferred_element_type=jnp.float32, precision=lax.Precision.HIGHEST).reshape(NO, _BN, Dh)
    ahi = jnp.dot(af, selT_ref[:, Dh:D],
                  preferred_element_type=jnp.float32, precision=lax.Precision.HIGHEST).reshape(NO, _BN, Dh)
    ctx_lo = jnp.sum(alo * (_lo(vw) + vpe[:, None, 0:Dh]), axis=0)
    ctx_hi = jnp.sum(ahi * (_hi(vw) + vpe[:, None, Dh:D]), axis=0)
    out = (jnp.dot(ctx_lo, Wo_ref[0:Dh, :],
                   preferred_element_type=jnp.float32, precision=lax.Precision.HIGHEST)
           + jnp.dot(ctx_hi, Wo_ref[Dh:D, :],
                     preferred_element_type=jnp.float32, precision=lax.Precision.HIGHEST)
           + bo_ref[...])
    enh = fp_ref[0] + out
    mu = jnp.mean(enh, axis=-1, keepdims=True)
    var = jnp.mean((enh - mu) ** 2, axis=-1, keepdims=True)
    out_ref[0] = (enh - mu) * lax.rsqrt(var + 1e-5) * g_ref[...] + be_ref[...]


def _stage_e(q, fp, nlinT, repT, kg, pepack, sel, selT, Wo, bo, gamma, beta):
    return pl.pallas_call(
        _e_body,
        grid=(B, N // _BN),
        in_specs=[
            pl.BlockSpec((1, _BN, D), lambda b, i: (b, i, 0)),
            pl.BlockSpec((1, _BN, D), lambda b, i: (b, i, 0)),
            pl.BlockSpec((1, NO, _BN), lambda b, i: (b, 0, i)),
            pl.BlockSpec((1, NO, _BN), lambda b, i: (b, 0, i)),
            pl.BlockSpec((1, NO, _BN, D), lambda b, i: (b, 0, i, 0)),
            pl.BlockSpec((2 * NO, D), lambda b, i: (0, 0)),
            pl.BlockSpec((D, H), lambda b, i: (0, 0)),
            pl.BlockSpec((H, D), lambda b, i: (0, 0)),
            pl.BlockSpec((D, D), lambda b, i: (0, 0)),
            pl.BlockSpec((1, D), lambda b, i: (0, 0)),
            pl.BlockSpec((1, D), lambda b, i: (0, 0)),
            pl.BlockSpec((1, D), lambda b, i: (0, 0)),
        ],
        out_specs=pl.BlockSpec((1, _BN, D), lambda b, i: (b, i, 0)),
        out_shape=jax.ShapeDtypeStruct((B, N, D), jnp.float32),
    )(q, fp, nlinT, repT, kg, pepack, sel, selT, Wo, bo.reshape(1, D),
      gamma.reshape(1, D), beta.reshape(1, D))


def kernel(features, coords, W_feat, b_feat, W_ia, b_ia, W_p1, b_p1, W_p2, b_p2,
           Wq, bq, Wk, bk, Wv, bv, Wo, bo, gamma, beta):
    offi = jnp.asarray(_OFFI)
    offsf = jnp.asarray(_OFFS32)
    sel = jnp.asarray(_SEL)
    selT = jnp.asarray(_SEL.T.copy())
    coordsT = coords.swapaxes(1, 2)
    fp, q, nlinT = _stage_a(features, coordsT, offi, W_feat, b_feat, Wq, bq)
    _, seg_f, repT_f = _stage_b(nlinT.reshape(B * NO * N))
    kvp, pepack = _stage_c(fp, seg_f.reshape(B, 1, N), offsf,
                           W_ia, b_ia, W_p1, b_p1, W_p2, b_p2, Wk, bk, Wv, bv)
    kg_f = _stage_d(kvp.reshape(B * N, D), repT_f)
    kg = kg_f.reshape(B, NO, N, D)
    repT = repT_f.reshape(B, NO, N)
    return _stage_e(q, fp, nlinT, repT, kg, pepack, sel, selT, Wo, bo, gamma, beta)


# speedup vs baseline: 1.0010x; 1.0010x over previous
import jax
import jax.numpy as jnp
import numpy as np
from jax import lax
from jax.experimental import pallas as pl
from jax.experimental.pallas import tpu as pltpu
from jax.experimental.pallas import tpu_sc as plsc

B, N = 2, 2048
D_IN, D = 128, 256
H, DH = 4, 64
GX, GY, GT = 128, 128, 200
TOT = GX * GY * GT
NO = 32
PT = N // 16
GSLC = TOT // 16

_OFF_LIST = [(dx, dy, dt) for dx in (-1, 0, 1) for dy in (-1, 0, 1) for dt in (-1, 0, 1)]
_OFFI = np.zeros((NO, 4), np.int32)
for _j, (_a, _b, _c) in enumerate(_OFF_LIST):
    _OFFI[_j] = (_a, _b, _c, 1)
_OFFS32 = np.zeros((NO, 3), np.float32)
for _j, (_a, _b, _c) in enumerate(_OFF_LIST):
    _OFFS32[_j] = (_a, _b, _c)
_SEL = np.zeros((D, H), np.float32)
for _h in range(H):
    _SEL[_h * DH:(_h + 1) * DH, _h] = 1.0


def _a_body(feat_ref, coordsT_ref, offi_ref, Wf_ref, bf_ref, Wq_ref, bq_ref,
            fp_ref, q_ref, nlinT_ref):
    f = feat_ref[0]
    fp = jnp.dot(f, Wf_ref[...], preferred_element_type=jnp.float32, precision=lax.Precision.HIGHEST) + bf_ref[...]
    fp_ref[0] = fp
    q_ref[0] = jnp.dot(fp, Wq_ref[...], preferred_element_type=jnp.float32, precision=lax.Precision.HIGHEST) + bq_ref[...]
    ct = coordsT_ref[0]
    vx = (jnp.clip(ct[0:1, :] * (1.0 / 256.0), 0.0, 1.0) * float(GX - 1)).astype(jnp.int32)
    vy = (jnp.clip(ct[1:2, :] * (1.0 / 256.0), 0.0, 1.0) * float(GY - 1)).astype(jnp.int32)
    vt = (jnp.clip(ct[2:3, :], 0.0, 1.0) * float(GT - 1)).astype(jnp.int32)
    nx = offi_ref[:, 0:1] + vx
    ny = offi_ref[:, 1:2] + vy
    nt = offi_ref[:, 2:3] + vt
    inb = ((nx >= 0) & (nx < GX) & (ny >= 0) & (ny < GY)
           & (nt >= 0) & (nt < GT) & (offi_ref[:, 3:4] > 0))
    nlin = nx * (GY * GT) + ny * GT + nt
    nlinT_ref[0] = jnp.where(inb, nlin, -1)


def _stage_a(features, coordsT, offi, W_feat, b_feat, Wq, bq):
    return pl.pallas_call(
        _a_body,
        grid=(B,),
        in_specs=[
            pl.BlockSpec((1, N, D_IN), lambda b: (b, 0, 0)),
            pl.BlockSpec((1, 3, N), lambda b: (b, 0, 0)),
            pl.BlockSpec((NO, 4), lambda b: (0, 0)),
            pl.BlockSpec((D_IN, D), lambda b: (0, 0)),
            pl.BlockSpec((1, D), lambda b: (0, 0)),
            pl.BlockSpec((D, D), lambda b: (0, 0)),
            pl.BlockSpec((1, D), lambda b: (0, 0)),
        ],
        out_specs=[
            pl.BlockSpec((1, N, D), lambda b: (b, 0, 0)),
            pl.BlockSpec((1, N, D), lambda b: (b, 0, 0)),
            pl.BlockSpec((1, NO, N), lambda b: (b, 0, 0)),
        ],
        out_shape=[
            jax.ShapeDtypeStruct((B, N, D), jnp.float32),
            jax.ShapeDtypeStruct((B, N, D), jnp.float32),
            jax.ShapeDtypeStruct((B, NO, N), jnp.int32),
        ],
    )(features, coordsT, offi, W_feat, b_feat.reshape(1, D), Wq, bq.reshape(1, D))


def _b1_body(nlinT_hbm, grid_hbm, fill_v, lin_v, val_v):
    c = lax.axis_index("c")
    t = lax.axis_index("s")
    i16 = lax.iota(jnp.int32, 16)
    gbase = c * TOT

    def _fill(i, _):
        fill_v[pl.ds(i * 16, 16)] = jnp.full((16,), -1, jnp.int32)
        return 0
    lax.fori_loop(0, 800, _fill, 0)

    def _small(g, _):
        val_v[pl.ds(g * 16, 16)] = t * PT + g * 16 + i16
        return 0
    lax.fori_loop(0, 8, _small, 0)

    def _memset(i, _):
        pltpu.sync_copy(fill_v, grid_hbm.at[pl.ds(gbase + t * GSLC + i * 12800, 12800)])
        return 0
    lax.fori_loop(0, 16, _memset, 0)

    pltpu.sync_copy(nlinT_hbm.at[pl.ds((c * NO + 13) * N + t * PT, PT)], lin_v)

    def _lclamp(g, _):
        lin_v[pl.ds(g * 16, 16)] = lin_v[pl.ds(g * 16, 16)] + gbase
        return 0
    lax.fori_loop(0, PT // 16, _lclamp, 0)

    plsc.subcore_barrier()

    pltpu.sync_copy(val_v, grid_hbm.at[lin_v])


def _b2_body(nlinT_hbm, grid_hbm, seg_hbm, repT_hbm,
             idxc_v, lin_v, seg_v, repc_v):
    c = lax.axis_index("c")
    t = lax.axis_index("s")
    gbase = c * TOT

    def _ldrow(j, _):
        pltpu.sync_copy(nlinT_hbm.at[pl.ds((c * NO + j) * N + t * PT, PT)], repc_v.at[j])
        return 0
    lax.fori_loop(0, NO, _ldrow, 0)
    pltpu.sync_copy(nlinT_hbm.at[pl.ds((c * NO + 13) * N + t * PT, PT)], lin_v)

    def _lclamp(g, _):
        lin_v[pl.ds(g * 16, 16)] = lin_v[pl.ds(g * 16, 16)] + gbase
        return 0
    lax.fori_loop(0, PT // 16, _lclamp, 0)

    def _clamp(i, _):
        idxc_v[i // 8, pl.ds((i % 8) * 16, 16)] = gbase + jnp.maximum(
            repc_v[i // 8, pl.ds((i % 8) * 16, 16)], 0)
        return 0
    lax.fori_loop(0, NO * PT // 16, _clamp, 0)

    pltpu.sync_copy(grid_hbm.at[lin_v], seg_v)
    pltpu.sync_copy(seg_v, seg_hbm.at[pl.ds(c * N + t * PT, PT)])

    def _ngather(j, _):
        pltpu.sync_copy(grid_hbm.at[idxc_v.at[j]], repc_v.at[j])
        return 0
    lax.fori_loop(0, NO, _ngather, 0)

    def _strow(j, _):
        pltpu.sync_copy(repc_v.at[j], repT_hbm.at[pl.ds((c * NO + j) * N + t * PT, PT)])
        return 0
    lax.fori_loop(0, NO, _strow, 0)


def _stage_b(nlinT_f):
    sc_mesh = plsc.VectorSubcoreMesh(core_axis_name="c", subcore_axis_name="s")
    b1 = pl.kernel(
        _b1_body,
        out_type=[jax.ShapeDtypeStruct((B * TOT,), jnp.int32)],
        mesh=sc_mesh,
        scratch_types=[
            pltpu.VMEM((12800,), jnp.int32),
            pltpu.VMEM((PT,), jnp.int32),
            pltpu.VMEM((PT,), jnp.int32),
        ],
    )
    grid = b1(nlinT_f)[0]
    b2 = pl.kernel(
        _b2_body,
        out_type=[
            jax.ShapeDtypeStruct((B * N,), jnp.int32),
            jax.ShapeDtypeStruct((B * NO * N,), jnp.int32),
        ],
        mesh=plsc.VectorSubcoreMesh(core_axis_name="c", subcore_axis_name="s"),
        scratch_types=[
            pltpu.VMEM((NO, PT), jnp.int32),
            pltpu.VMEM((PT,), jnp.int32),
            pltpu.VMEM((PT,), jnp.int32),
            pltpu.VMEM((NO, PT), jnp.int32),
        ],
    )
    seg_f, repT_f = b2(nlinT_f, grid)
    return grid, seg_f, repT_f


_SN = 256


def _c_body(fp_ref, seg_ref, offs_ref, Wia_ref, bia_ref, Wp1_ref, bp1_ref,
            Wp2_ref, bp2_ref, Wk_ref, bk_ref, Wv_ref, bv_ref,
            kvp_ref, pe_ref):
    i = pl.program_id(1)
    seg = seg_ref[0]
    uio = lax.broadcasted_iota(jnp.int32, (_SN, 1), 0) + i * _SN
    pmat = (uio == seg).astype(jnp.float32)
    cnt = jnp.maximum(jnp.sum(pmat, axis=1, keepdims=True), 1.0)
    sums = jnp.dot(pmat, fp_ref[0], preferred_element_type=jnp.float32, precision=lax.Precision.HIGHEST)
    means = sums * (1.0 / cnt)
    vf = jnp.maximum(
        jnp.dot(means, Wia_ref[...], preferred_element_type=jnp.float32, precision=lax.Precision.HIGHEST) + bia_ref[...], 0.0)
    kvf = jnp.dot(vf, Wk_ref[...], preferred_element_type=jnp.float32, precision=lax.Precision.HIGHEST)
    vvf = jnp.dot(vf, Wv_ref[...], preferred_element_type=jnp.float32, precision=lax.Precision.HIGHEST)

    def _pack(x):
        lo = lax.bitcast_convert_type(
            x[:, 0:D // 2].astype(jnp.bfloat16).astype(jnp.float32), jnp.int32)
        hi = lax.bitcast_convert_type(
            x[:, D // 2:D].astype(jnp.bfloat16).astype(jnp.float32), jnp.int32)
        return hi | ((lo >> 16) & 0xffff)
    kvp_ref[0] = jnp.concatenate([_pack(kvf), _pack(vvf)], axis=1)
    offs = offs_ref[...]
    pe_h = jnp.maximum(jnp.dot(offs, Wp1_ref[...], preferred_element_type=jnp.float32, precision=lax.Precision.HIGHEST)
                       + bp1_ref[...], 0.0)
    pe = jnp.dot(pe_h, Wp2_ref[...], preferred_element_type=jnp.float32, precision=lax.Precision.HIGHEST) + bp2_ref[...]
    kpe = jnp.dot(pe, Wk_ref[...], preferred_element_type=jnp.float32, precision=lax.Precision.HIGHEST) + bk_ref[...]
    vpe = jnp.dot(pe, Wv_ref[...], preferred_element_type=jnp.float32, precision=lax.Precision.HIGHEST) + bv_ref[...]
    pe_ref[...] = jnp.concatenate([kpe, vpe], axis=0)


def _stage_c(fp, seg2, offsf, W_ia, b_ia, W_p1, b_p1, W_p2, b_p2, Wk, bk, Wv, bv):
    return pl.pallas_call(
        _c_body,
        grid=(B, N // _SN),
        in_specs=[
            pl.BlockSpec((1, N, D), lambda b, i: (b, 0, 0)),
            pl.BlockSpec((1, 1, N), lambda b, i: (b, 0, 0)),
            pl.BlockSpec((NO, 3), lambda b, i: (0, 0)),
            pl.BlockSpec((D, D), lambda b, i: (0, 0)),
            pl.BlockSpec((1, D), lambda b, i: (0, 0)),
            pl.BlockSpec((3, D // 2), lambda b, i: (0, 0)),
            pl.BlockSpec((1, D // 2), lambda b, i: (0, 0)),
            pl.BlockSpec((D // 2, D), lambda b, i: (0, 0)),
            pl.BlockSpec((1, D), lambda b, i: (0, 0)),
            pl.BlockSpec((D, D), lambda b, i: (0, 0)),
            pl.BlockSpec((1, D), lambda b, i: (0, 0)),
            pl.BlockSpec((D, D), lambda b, i: (0, 0)),
            pl.BlockSpec((1, D), lambda b, i: (0, 0)),
        ],
        out_specs=[
            pl.BlockSpec((1, _SN, D), lambda b, i: (b, i, 0)),
            pl.BlockSpec((2 * NO, D), lambda b, i: (0, 0)),
        ],
        out_shape=[
            jax.ShapeDtypeStruct((B, N, D), jnp.int32),
            jax.ShapeDtypeStruct((2 * NO, D), jnp.float32),
        ],
    )(fp, seg2, offsf, W_ia, b_ia.reshape(1, D), W_p1, b_p1.reshape(1, D // 2),
      W_p2, b_p2.reshape(1, D), Wk, bk.reshape(1, D), Wv, bv.reshape(1, D))


_DC = PT
_NCH = 27


def _d_body(kvp_hbm, repT_hbm, kg_hbm,
            raw_v, repc_v, kb00, kb01, kb10, kb11,
            kg0, kg1, kg2, kg3, kw0, kw1, kw2, kw3):
    c = lax.axis_index("c")
    t = lax.axis_index("s")
    kbufs = ((kb00, kb01), (kb10, kb11))
    kgs = ((kg0, kg2), (kg1, kg3))
    kws = ((kw0, kw2), (kw1, kw3))

    def _ldrow(j, _):
        pltpu.sync_copy(repT_hbm.at[pl.ds((c * NO + j) * N + t * PT, PT)], raw_v.at[j])
        return 0
    lax.fori_loop(0, _NCH, _ldrow, 0)

    def _clamp(i, _):
        repc_v[pl.ds(i * 16, 16)] = c * N + jnp.maximum(raw_v[i // 8, pl.ds((i % 8) * 16, 16)], 0)
        return 0
    lax.fori_loop(0, _NCH * PT // 16, _clamp, 0)

    _HD = _DC // 2

    def _startg(i, p):
        for s in range(2):
            idx = repc_v.at[pl.ds(i * _DC + s * _HD, _HD)]
            pltpu.async_copy(kvp_hbm.at[idx], kbufs[p][s], kgs[p][s])

    def _waitg(i, p):
        for s in range(2):
            idx = repc_v.at[pl.ds(i * _DC + s * _HD, _HD)]
            pltpu.make_async_copy(kvp_hbm.at[idx], kbufs[p][s], kgs[p][s]).wait()

    def _out(i):
        return (c * NO + i) * N + t * PT

    def _startw(i, p):
        for s in range(2):
            pltpu.async_copy(kbufs[p][s], kg_hbm.at[pl.ds(_out(i) + s * _HD, _HD), :], kws[p][s])

    def _drainw(p):
        for s in range(2):
            pltpu.make_async_copy(kbufs[p][s], kg_hbm.at[pl.ds(0, _HD), :], kws[p][s]).wait()

    _startg(0, 0)

    def _pair(g, _):
        for p in (0, 1):
            i = 2 * g + p
            q = 1 - p
            if p == 0:
                @pl.when(g >= 1)
                def _():
                    _drainw(q)
                _startg(i + 1, q)
            else:
                _drainw(q)
                _startg(i + 1, q)
            _waitg(i, p)
            _startw(i, p)
        return 0
    lax.fori_loop(0, (_NCH - 1) // 2, _pair, 0)

    _waitg(_NCH - 1, 0)
    _startw(_NCH - 1, 0)
    _drainw(0)
    _drainw(1)


def _stage_d(kvp_f, repT_f):
    f = pl.kernel(
        _d_body,
        out_type=[
            jax.ShapeDtypeStruct((B * NO * N, D), jnp.int32),
        ],
        mesh=plsc.VectorSubcoreMesh(core_axis_name="c", subcore_axis_name="s"),
        scratch_types=[
            pltpu.VMEM((NO, PT), jnp.int32),
            pltpu.VMEM((NO * PT,), jnp.int32),
            pltpu.VMEM((_DC // 2, D), jnp.int32),
            pltpu.VMEM((_DC // 2, D), jnp.int32),
            pltpu.VMEM((_DC // 2, D), jnp.int32),
            pltpu.VMEM((_DC // 2, D), jnp.int32),
            pltpu.SemaphoreType.DMA,
            pltpu.SemaphoreType.DMA,
            pltpu.SemaphoreType.DMA,
            pltpu.SemaphoreType.DMA,
            pltpu.SemaphoreType.DMA,
            pltpu.SemaphoreType.DMA,
            pltpu.SemaphoreType.DMA,
            pltpu.SemaphoreType.DMA,
        ],
    )
    return f(kvp_f, repT_f)[0]


_BN = 128


def _e_body(q_ref, fp_ref, nlinT_ref, repT_ref, kg_ref, pe_ref,
            sel_ref, selT_ref, Wo_ref, bo_ref, g_ref, be_ref, out_ref):
    q = q_ref[0]
    found = (nlinT_ref[0] >= 0) & (repT_ref[0] >= 0)
    pen = jnp.where(found, 0.0, -1e9)
    kpe = pe_ref[0:NO, :]
    vpe = pe_ref[NO:2 * NO, :]
    Dh = D // 2
    validm = (lax.broadcasted_iota(jnp.int32, (NO, 1), 0) < 27).astype(jnp.int32)
    kvw = kg_ref[0] * validm[:, :, None]
    kw = kvw[:, :, 0:Dh]
    vw = kvw[:, :, Dh:D]

    def _lo(w):
        return lax.bitcast_convert_type(w << 16, jnp.float32)

    def _hi(w):
        return lax.bitcast_convert_type(w & jnp.int32(-65536), jnp.float32)

    klo = _lo(kw) + kpe[:, None, 0:Dh]
    khi = _hi(kw) + kpe[:, None, Dh:D]
    plo = q[None, :, 0:Dh] * klo
    phi = q[None, :, Dh:D] * khi
    logits = (jnp.dot(plo.reshape(NO * _BN, Dh), sel_ref[0:Dh, :],
                      preferred_element_type=jnp.float32, precision=lax.Precision.HIGHEST)
              + jnp.dot(phi.reshape(NO * _BN, Dh), sel_ref[Dh:D, :],
                        preferred_element_type=jnp.float32, precision=lax.Precision.HIGHEST)
              ).reshape(NO, _BN, H)
    logits = logits * (1.0 / np.sqrt(DH).astype(np.float32)) + pen[:, :, None]
    m = jnp.max(logits, axis=0, keepdims=True)
    p = jnp.exp(logits - m)
    attn = p / jnp.sum(p, axis=0, keepdims=True)
    af = attn.reshape(NO * _BN, H)
    alo = jnp.dot(af, selT_ref[:, 0:Dh],
                  preferred_element_type=jnp.float32, precision=lax.Precision.HIGHEST).reshape(NO, _BN, Dh)
    ahi = jnp.dot(af, selT_ref[:, Dh:D],
                  preferred_element_type=jnp.float32, precision=lax.Precision.HIGHEST).reshape(NO, _BN, Dh)
    ctx_lo = jnp.sum(alo * (_lo(vw) + vpe[:, None, 0:Dh]), axis=0)
    ctx_hi = jnp.sum(ahi * (_hi(vw) + vpe[:, None, Dh:D]), axis=0)
    out = (jnp.dot(ctx_lo, Wo_ref[0:Dh, :],
                   preferred_element_type=jnp.float32, precision=lax.Precision.HIGHEST)
           + jnp.dot(ctx_hi, Wo_ref[Dh:D, :],
                     preferred_element_type=jnp.float32, precision=lax.Precision.HIGHEST)
           + bo_ref[...])
    enh = fp_ref[0] + out
    mu = jnp.mean(enh, axis=-1, keepdims=True)
    var = jnp.mean((enh - mu) ** 2, axis=-1, keepdims=True)
    out_ref[0] = (enh - mu) * lax.rsqrt(var + 1e-5) * g_ref[...] + be_ref[...]


def _stage_e(q, fp, nlinT, repT, kg, pepack, sel, selT, Wo, bo, gamma, beta):
    return pl.pallas_call(
        _e_body,
        grid=(B, N // _BN),
        in_specs=[
            pl.BlockSpec((1, _BN, D), lambda b, i: (b, i, 0)),
            pl.BlockSpec((1, _BN, D), lambda b, i: (b, i, 0)),
            pl.BlockSpec((1, NO, _BN), lambda b, i: (b, 0, i)),
            pl.BlockSpec((1, NO, _BN), lambda b, i: (b, 0, i)),
            pl.BlockSpec((1, NO, _BN, D), lambda b, i: (b, 0, i, 0)),
            pl.BlockSpec((2 * NO, D), lambda b, i: (0, 0)),
            pl.BlockSpec((D, H), lambda b, i: (0, 0)),
            pl.BlockSpec((H, D), lambda b, i: (0, 0)),
            pl.BlockSpec((D, D), lambda b, i: (0, 0)),
            pl.BlockSpec((1, D), lambda b, i: (0, 0)),
            pl.BlockSpec((1, D), lambda b, i: (0, 0)),
            pl.BlockSpec((1, D), lambda b, i: (0, 0)),
        ],
        out_specs=pl.BlockSpec((1, _BN, D), lambda b, i: (b, i, 0)),
        out_shape=jax.ShapeDtypeStruct((B, N, D), jnp.float32),
    )(q, fp, nlinT, repT, kg, pepack, sel, selT, Wo, bo.reshape(1, D),
      gamma.reshape(1, D), beta.reshape(1, D))


def kernel(features, coords, W_feat, b_feat, W_ia, b_ia, W_p1, b_p1, W_p2, b_p2,
           Wq, bq, Wk, bk, Wv, bv, Wo, bo, gamma, beta):
    offi = jnp.asarray(_OFFI)
    offsf = jnp.asarray(_OFFS32)
    sel = jnp.asarray(_SEL)
    selT = jnp.asarray(_SEL.T.copy())
    coordsT = coords.swapaxes(1, 2)
    fp, q, nlinT = _stage_a(features, coordsT, offi, W_feat, b_feat, Wq, bq)
    _, seg_f, repT_f = _stage_b(nlinT.reshape(B * NO * N))
    kvp, pepack = _stage_c(fp, seg_f.reshape(B, 1, N), offsf,
                           W_ia, b_ia, W_p1, b_p1, W_p2, b_p2, Wk, bk, Wv, bv)
    kg_f = _stage_d(kvp.reshape(B * N, D), repT_f)
    kg = kg_f.reshape(B, NO, N, D)
    repT = repT_f.reshape(B, NO, N)
    return _stage_e(q, fp, nlinT, repT, kg, pepack, sel, selT, Wo, bo, gamma, beta)


# speedup vs baseline: 1.0182x; 1.0172x over previous
import jax
import jax.numpy as jnp
import numpy as np
from jax import lax
from jax.experimental import pallas as pl
from jax.experimental.pallas import tpu as pltpu
from jax.experimental.pallas import tpu_sc as plsc

B, N = 2, 2048
D_IN, D = 128, 256
H, DH = 4, 64
GX, GY, GT = 128, 128, 200
TOT = GX * GY * GT
NO = 32
PT = N // 16
GSLC = TOT // 16

_OFF_LIST = [(dx, dy, dt) for dx in (-1, 0, 1) for dy in (-1, 0, 1) for dt in (-1, 0, 1)]
_OFFI = np.zeros((NO, 4), np.int32)
for _j, (_a, _b, _c) in enumerate(_OFF_LIST):
    _OFFI[_j] = (_a, _b, _c, 1)
_OFFS32 = np.zeros((NO, 3), np.float32)
for _j, (_a, _b, _c) in enumerate(_OFF_LIST):
    _OFFS32[_j] = (_a, _b, _c)
_SEL = np.zeros((D, H), np.float32)
for _h in range(H):
    _SEL[_h * DH:(_h + 1) * DH, _h] = 1.0


def _a_body(feat_ref, coordsT_ref, offi_ref, Wf_ref, bf_ref, Wq_ref, bq_ref,
            fp_ref, q_ref, nlinT_ref):
    f = feat_ref[0]
    fp = jnp.dot(f, Wf_ref[...], preferred_element_type=jnp.float32, precision=lax.Precision.HIGHEST) + bf_ref[...]
    fp_ref[0] = fp
    q_ref[0] = jnp.dot(fp, Wq_ref[...], preferred_element_type=jnp.float32, precision=lax.Precision.HIGHEST) + bq_ref[...]
    ct = coordsT_ref[0]
    vx = (jnp.clip(ct[0:1, :] * (1.0 / 256.0), 0.0, 1.0) * float(GX - 1)).astype(jnp.int32)
    vy = (jnp.clip(ct[1:2, :] * (1.0 / 256.0), 0.0, 1.0) * float(GY - 1)).astype(jnp.int32)
    vt = (jnp.clip(ct[2:3, :], 0.0, 1.0) * float(GT - 1)).astype(jnp.int32)
    nx = offi_ref[:, 0:1] + vx
    ny = offi_ref[:, 1:2] + vy
    nt = offi_ref[:, 2:3] + vt
    inb = ((nx >= 0) & (nx < GX) & (ny >= 0) & (ny < GY)
           & (nt >= 0) & (nt < GT) & (offi_ref[:, 3:4] > 0))
    nlin = nx * (GY * GT) + ny * GT + nt
    nlinT_ref[0] = jnp.where(inb, nlin, -1)


def _stage_a(features, coordsT, offi, W_feat, b_feat, Wq, bq):
    return pl.pallas_call(
        _a_body,
        grid=(B,),
        in_specs=[
            pl.BlockSpec((1, N, D_IN), lambda b: (b, 0, 0)),
            pl.BlockSpec((1, 3, N), lambda b: (b, 0, 0)),
            pl.BlockSpec((NO, 4), lambda b: (0, 0)),
            pl.BlockSpec((D_IN, D), lambda b: (0, 0)),
            pl.BlockSpec((1, D), lambda b: (0, 0)),
            pl.BlockSpec((D, D), lambda b: (0, 0)),
            pl.BlockSpec((1, D), lambda b: (0, 0)),
        ],
        out_specs=[
            pl.BlockSpec((1, N, D), lambda b: (b, 0, 0)),
            pl.BlockSpec((1, N, D), lambda b: (b, 0, 0)),
            pl.BlockSpec((1, NO, N), lambda b: (b, 0, 0)),
        ],
        out_shape=[
            jax.ShapeDtypeStruct((B, N, D), jnp.float32),
            jax.ShapeDtypeStruct((B, N, D), jnp.float32),
            jax.ShapeDtypeStruct((B, NO, N), jnp.int32),
        ],
    )(features, coordsT, offi, W_feat, b_feat.reshape(1, D), Wq, bq.reshape(1, D))


def _b1_body(nlinT_hbm, grid_hbm, fill_v, lin_v, val_v):
    c = lax.axis_index("c")
    t = lax.axis_index("s")
    i16 = lax.iota(jnp.int32, 16)
    gbase = c * TOT

    def _fill(i, _):
        fill_v[pl.ds(i * 16, 16)] = jnp.full((16,), -1, jnp.int32)
        return 0
    lax.fori_loop(0, 800, _fill, 0)

    def _small(g, _):
        val_v[pl.ds(g * 16, 16)] = t * PT + g * 16 + i16
        return 0
    lax.fori_loop(0, 8, _small, 0)

    def _memset(i, _):
        pltpu.sync_copy(fill_v, grid_hbm.at[pl.ds(gbase + t * GSLC + i * 12800, 12800)])
        return 0
    lax.fori_loop(0, 16, _memset, 0)

    pltpu.sync_copy(nlinT_hbm.at[pl.ds((c * NO + 13) * N + t * PT, PT)], lin_v)

    def _lclamp(g, _):
        lin_v[pl.ds(g * 16, 16)] = lin_v[pl.ds(g * 16, 16)] + gbase
        return 0
    lax.fori_loop(0, PT // 16, _lclamp, 0)

    plsc.subcore_barrier()

    pltpu.sync_copy(val_v, grid_hbm.at[lin_v])


def _b2_body(nlinT_hbm, grid_hbm, seg_hbm, repT_hbm,
             idxc_v, lin_v, seg_v, repc_v):
    c = lax.axis_index("c")
    t = lax.axis_index("s")
    gbase = c * TOT

    def _ldrow(j, _):
        pltpu.sync_copy(nlinT_hbm.at[pl.ds((c * NO + j) * N + t * PT, PT)], repc_v.at[j])
        return 0
    lax.fori_loop(0, 27, _ldrow, 0)
    pltpu.sync_copy(nlinT_hbm.at[pl.ds((c * NO + 13) * N + t * PT, PT)], lin_v)

    def _lclamp(g, _):
        lin_v[pl.ds(g * 16, 16)] = lin_v[pl.ds(g * 16, 16)] + gbase
        return 0
    lax.fori_loop(0, PT // 16, _lclamp, 0)

    def _clamp(i, _):
        idxc_v[i // 8, pl.ds((i % 8) * 16, 16)] = gbase + jnp.maximum(
            repc_v[i // 8, pl.ds((i % 8) * 16, 16)], 0)
        return 0
    lax.fori_loop(0, 27 * PT // 16, _clamp, 0)

    pltpu.sync_copy(grid_hbm.at[lin_v], seg_v)
    pltpu.sync_copy(seg_v, seg_hbm.at[pl.ds(c * N + t * PT, PT)])

    def _ngather(j, _):
        pltpu.sync_copy(grid_hbm.at[idxc_v.at[j]], repc_v.at[j])
        return 0
    lax.fori_loop(0, 27, _ngather, 0)

    def _strow(j, _):
        pltpu.sync_copy(repc_v.at[j], repT_hbm.at[pl.ds((c * NO + j) * N + t * PT, PT)])
        return 0
    lax.fori_loop(0, 27, _strow, 0)


def _stage_b(nlinT_f):
    sc_mesh = plsc.VectorSubcoreMesh(core_axis_name="c", subcore_axis_name="s")
    b1 = pl.kernel(
        _b1_body,
        out_type=[jax.ShapeDtypeStruct((B * TOT,), jnp.int32)],
        mesh=sc_mesh,
        scratch_types=[
            pltpu.VMEM((12800,), jnp.int32),
            pltpu.VMEM((PT,), jnp.int32),
            pltpu.VMEM((PT,), jnp.int32),
        ],
    )
    grid = b1(nlinT_f)[0]
    b2 = pl.kernel(
        _b2_body,
        out_type=[
            jax.ShapeDtypeStruct((B * N,), jnp.int32),
            jax.ShapeDtypeStruct((B * NO * N,), jnp.int32),
        ],
        mesh=plsc.VectorSubcoreMesh(core_axis_name="c", subcore_axis_name="s"),
        scratch_types=[
            pltpu.VMEM((NO, PT), jnp.int32),
            pltpu.VMEM((PT,), jnp.int32),
            pltpu.VMEM((PT,), jnp.int32),
            pltpu.VMEM((NO, PT), jnp.int32),
        ],
    )
    seg_f, repT_f = b2(nlinT_f, grid)
    return grid, seg_f, repT_f


_SN = 256


def _c_body(fp_ref, seg_ref, offs_ref, Wia_ref, bia_ref, Wp1_ref, bp1_ref,
            Wp2_ref, bp2_ref, Wk_ref, bk_ref, Wv_ref, bv_ref,
            kvp_ref, pe_ref):
    i = pl.program_id(1)
    seg = seg_ref[0]
    uio = lax.broadcasted_iota(jnp.int32, (_SN, 1), 0) + i * _SN
    pmat = (uio == seg).astype(jnp.float32)
    cnt = jnp.maximum(jnp.sum(pmat, axis=1, keepdims=True), 1.0)
    sums = jnp.dot(pmat, fp_ref[0], preferred_element_type=jnp.float32, precision=lax.Precision.HIGHEST)
    means = sums * (1.0 / cnt)
    vf = jnp.maximum(
        jnp.dot(means, Wia_ref[...], preferred_element_type=jnp.float32, precision=lax.Precision.HIGHEST) + bia_ref[...], 0.0)
    kvf = jnp.dot(vf, Wk_ref[...], preferred_element_type=jnp.float32, precision=lax.Precision.HIGHEST)
    vvf = jnp.dot(vf, Wv_ref[...], preferred_element_type=jnp.float32, precision=lax.Precision.HIGHEST)

    def _pack(x):
        lo = lax.bitcast_convert_type(
            x[:, 0:D // 2].astype(jnp.bfloat16).astype(jnp.float32), jnp.int32)
        hi = lax.bitcast_convert_type(
            x[:, D // 2:D].astype(jnp.bfloat16).astype(jnp.float32), jnp.int32)
        return hi | ((lo >> 16) & 0xffff)
    kvp_ref[0] = jnp.concatenate([_pack(kvf), _pack(vvf)], axis=1)
    offs = offs_ref[...]
    pe_h = jnp.maximum(jnp.dot(offs, Wp1_ref[...], preferred_element_type=jnp.float32, precision=lax.Precision.HIGHEST)
                       + bp1_ref[...], 0.0)
    pe = jnp.dot(pe_h, Wp2_ref[...], preferred_element_type=jnp.float32, precision=lax.Precision.HIGHEST) + bp2_ref[...]
    kpe = jnp.dot(pe, Wk_ref[...], preferred_element_type=jnp.float32, precision=lax.Precision.HIGHEST) + bk_ref[...]
    vpe = jnp.dot(pe, Wv_ref[...], preferred_element_type=jnp.float32, precision=lax.Precision.HIGHEST) + bv_ref[...]
    pe_ref[...] = jnp.concatenate([kpe, vpe], axis=0)


def _stage_c(fp, seg2, offsf, W_ia, b_ia, W_p1, b_p1, W_p2, b_p2, Wk, bk, Wv, bv):
    return pl.pallas_call(
        _c_body,
        grid=(B, N // _SN),
        in_specs=[
            pl.BlockSpec((1, N, D), lambda b, i: (b, 0, 0)),
            pl.BlockSpec((1, 1, N), lambda b, i: (b, 0, 0)),
            pl.BlockSpec((NO, 3), lambda b, i: (0, 0)),
            pl.BlockSpec((D, D), lambda b, i: (0, 0)),
            pl.BlockSpec((1, D), lambda b, i: (0, 0)),
            pl.BlockSpec((3, D // 2), lambda b, i: (0, 0)),
            pl.BlockSpec((1, D // 2), lambda b, i: (0, 0)),
            pl.BlockSpec((D // 2, D), lambda b, i: (0, 0)),
            pl.BlockSpec((1, D), lambda b, i: (0, 0)),
            pl.BlockSpec((D, D), lambda b, i: (0, 0)),
            pl.BlockSpec((1, D), lambda b, i: (0, 0)),
            pl.BlockSpec((D, D), lambda b, i: (0, 0)),
            pl.BlockSpec((1, D), lambda b, i: (0, 0)),
        ],
        out_specs=[
            pl.BlockSpec((1, _SN, D), lambda b, i: (b, i, 0)),
            pl.BlockSpec((2 * NO, D), lambda b, i: (0, 0)),
        ],
        out_shape=[
            jax.ShapeDtypeStruct((B, N, D), jnp.int32),
            jax.ShapeDtypeStruct((2 * NO, D), jnp.float32),
        ],
    )(fp, seg2, offsf, W_ia, b_ia.reshape(1, D), W_p1, b_p1.reshape(1, D // 2),
      W_p2, b_p2.reshape(1, D), Wk, bk.reshape(1, D), Wv, bv.reshape(1, D))


_DC = PT
_NCH = 27


def _d_body(kvp_hbm, repT_hbm, kg_hbm,
            raw_v, repc_v, kb00, kb01, kb10, kb11,
            kg0, kg1, kg2, kg3, kw0, kw1, kw2, kw3):
    c = lax.axis_index("c")
    t = lax.axis_index("s")
    kbufs = ((kb00, kb01), (kb10, kb11))
    kgs = ((kg0, kg2), (kg1, kg3))
    kws = ((kw0, kw2), (kw1, kw3))

    def _ldrow(j, _):
        pltpu.sync_copy(repT_hbm.at[pl.ds((c * NO + j) * N + t * PT, PT)], raw_v.at[j])
        return 0
    lax.fori_loop(0, _NCH, _ldrow, 0)

    def _clamp(i, _):
        repc_v[pl.ds(i * 16, 16)] = c * N + jnp.maximum(raw_v[i // 8, pl.ds((i % 8) * 16, 16)], 0)
        return 0
    lax.fori_loop(0, _NCH * PT // 16, _clamp, 0)

    _HD = _DC // 2

    def _startg(i, p):
        for s in range(2):
            idx = repc_v.at[pl.ds(i * _DC + s * _HD, _HD)]
            pltpu.async_copy(kvp_hbm.at[idx], kbufs[p][s], kgs[p][s])

    def _waitg(i, p):
        for s in range(2):
            idx = repc_v.at[pl.ds(i * _DC + s * _HD, _HD)]
            pltpu.make_async_copy(kvp_hbm.at[idx], kbufs[p][s], kgs[p][s]).wait()

    def _out(i):
        return (c * NO + i) * N + t * PT

    def _startw(i, p):
        for s in range(2):
            pltpu.async_copy(kbufs[p][s], kg_hbm.at[pl.ds(_out(i) + s * _HD, _HD), :], kws[p][s])

    def _drainw(p):
        for s in range(2):
            pltpu.make_async_copy(kbufs[p][s], kg_hbm.at[pl.ds(0, _HD), :], kws[p][s]).wait()

    _startg(0, 0)

    def _pair(g, _):
        for p in (0, 1):
            i = 2 * g + p
            q = 1 - p
            if p == 0:
                @pl.when(g >= 1)
                def _():
                    _drainw(q)
                _startg(i + 1, q)
            else:
                _drainw(q)
                _startg(i + 1, q)
            _waitg(i, p)
            _startw(i, p)
        return 0
    lax.fori_loop(0, (_NCH - 1) // 2, _pair, 0)

    _waitg(_NCH - 1, 0)
    _startw(_NCH - 1, 0)
    _drainw(0)
    _drainw(1)


def _stage_d(kvp_f, repT_f):
    f = pl.kernel(
        _d_body,
        out_type=[
            jax.ShapeDtypeStruct((B * NO * N, D), jnp.int32),
        ],
        mesh=plsc.VectorSubcoreMesh(core_axis_name="c", subcore_axis_name="s"),
        scratch_types=[
            pltpu.VMEM((NO, PT), jnp.int32),
            pltpu.VMEM((NO * PT,), jnp.int32),
            pltpu.VMEM((_DC // 2, D), jnp.int32),
            pltpu.VMEM((_DC // 2, D), jnp.int32),
            pltpu.VMEM((_DC // 2, D), jnp.int32),
            pltpu.VMEM((_DC // 2, D), jnp.int32),
            pltpu.SemaphoreType.DMA,
            pltpu.SemaphoreType.DMA,
            pltpu.SemaphoreType.DMA,
            pltpu.SemaphoreType.DMA,
            pltpu.SemaphoreType.DMA,
            pltpu.SemaphoreType.DMA,
            pltpu.SemaphoreType.DMA,
            pltpu.SemaphoreType.DMA,
        ],
    )
    return f(kvp_f, repT_f)[0]


_BN = 128


def _e_body(q_ref, fp_ref, nlinT_ref, repT_ref, kg_ref, pe_ref,
            sel_ref, selT_ref, Wo_ref, bo_ref, g_ref, be_ref, out_ref):
    q = q_ref[0]
    found = (nlinT_ref[0] >= 0) & (repT_ref[0] >= 0)
    pen = jnp.where(found, 0.0, -1e9)
    kpe = pe_ref[0:NO, :]
    vpe = pe_ref[NO:2 * NO, :]
    Dh = D // 2
    validm = (lax.broadcasted_iota(jnp.int32, (NO, 1), 0) < 27).astype(jnp.int32)
    kvw = kg_ref[0] * validm[:, :, None]
    kw = kvw[:, :, 0:Dh]
    vw = kvw[:, :, Dh:D]

    def _lo(w):
        return lax.bitcast_convert_type(w << 16, jnp.float32)

    def _hi(w):
        return lax.bitcast_convert_type(w & jnp.int32(-65536), jnp.float32)

    klo = _lo(kw) + kpe[:, None, 0:Dh]
    khi = _hi(kw) + kpe[:, None, Dh:D]
    plo = q[None, :, 0:Dh] * klo
    phi = q[None, :, Dh:D] * khi
    logits = (jnp.dot(plo.reshape(NO * _BN, Dh), sel_ref[0:Dh, :],
                      preferred_element_type=jnp.float32, precision=lax.Precision.HIGHEST)
              + jnp.dot(phi.reshape(NO * _BN, Dh), sel_ref[Dh:D, :],
                        preferred_element_type=jnp.float32, precision=lax.Precision.HIGHEST)
              ).reshape(NO, _BN, H)
    logits = logits * (1.0 / np.sqrt(DH).astype(np.float32)) + pen[:, :, None]
    m = jnp.max(logits, axis=0, keepdims=True)
    p = jnp.exp(logits - m)
    attn = p / jnp.sum(p, axis=0, keepdims=True)
    af = attn.reshape(NO * _BN, H)
    alo = jnp.dot(af, selT_ref[:, 0:Dh],
                  preferred_element_type=jnp.float32, precision=lax.Precision.HIGHEST).reshape(NO, _BN, Dh)
    ahi = jnp.dot(af, selT_ref[:, Dh:D],
                  preferred_element_type=jnp.float32, precision=lax.Precision.HIGHEST).reshape(NO, _BN, Dh)
    ctx_lo = jnp.sum(alo * (_lo(vw) + vpe[:, None, 0:Dh]), axis=0)
    ctx_hi = jnp.sum(ahi * (_hi(vw) + vpe[:, None, Dh:D]), axis=0)
    out = (jnp.dot(ctx_lo, Wo_ref[0:Dh, :],
                   preferred_element_type=jnp.float32, precision=lax.Precision.HIGHEST)
           + jnp.dot(ctx_hi, Wo_ref[Dh:D, :],
                     preferred_element_type=jnp.float32, precision=lax.Precision.HIGHEST)
           + bo_ref[...])
    enh = fp_ref[0] + out
    mu = jnp.mean(enh, axis=-1, keepdims=True)
    var = jnp.mean((enh - mu) ** 2, axis=-1, keepdims=True)
    out_ref[0] = (enh - mu) * lax.rsqrt(var + 1e-5) * g_ref[...] + be_ref[...]


def _stage_e(q, fp, nlinT, repT, kg, pepack, sel, selT, Wo, bo, gamma, beta):
    return pl.pallas_call(
        _e_body,
        grid=(B, N // _BN),
        in_specs=[
            pl.BlockSpec((1, _BN, D), lambda b, i: (b, i, 0)),
            pl.BlockSpec((1, _BN, D), lambda b, i: (b, i, 0)),
            pl.BlockSpec((1, NO, _BN), lambda b, i: (b, 0, i)),
            pl.BlockSpec((1, NO, _BN), lambda b, i: (b, 0, i)),
            pl.BlockSpec((1, NO, _BN, D), lambda b, i: (b, 0, i, 0)),
            pl.BlockSpec((2 * NO, D), lambda b, i: (0, 0)),
            pl.BlockSpec((D, H), lambda b, i: (0, 0)),
            pl.BlockSpec((H, D), lambda b, i: (0, 0)),
            pl.BlockSpec((D, D), lambda b, i: (0, 0)),
            pl.BlockSpec((1, D), lambda b, i: (0, 0)),
            pl.BlockSpec((1, D), lambda b, i: (0, 0)),
            pl.BlockSpec((1, D), lambda b, i: (0, 0)),
        ],
        out_specs=pl.BlockSpec((1, _BN, D), lambda b, i: (b, i, 0)),
        out_shape=jax.ShapeDtypeStruct((B, N, D), jnp.float32),
    )(q, fp, nlinT, repT, kg, pepack, sel, selT, Wo, bo.reshape(1, D),
      gamma.reshape(1, D), beta.reshape(1, D))


def kernel(features, coords, W_feat, b_feat, W_ia, b_ia, W_p1, b_p1, W_p2, b_p2,
           Wq, bq, Wk, bk, Wv, bv, Wo, bo, gamma, beta):
    offi = jnp.asarray(_OFFI)
    offsf = jnp.asarray(_OFFS32)
    sel = jnp.asarray(_SEL)
    selT = jnp.asarray(_SEL.T.copy())
    coordsT = coords.swapaxes(1, 2)
    fp, q, nlinT = _stage_a(features, coordsT, offi, W_feat, b_feat, Wq, bq)
    _, seg_f, repT_f = _stage_b(nlinT.reshape(B * NO * N))
    kvp, pepack = _stage_c(fp, seg_f.reshape(B, 1, N), offsf,
                           W_ia, b_ia, W_p1, b_p1, W_p2, b_p2, Wk, bk, Wv, bv)
    kg_f = _stage_d(kvp.reshape(B * N, D), repT_f)
    kg = kg_f.reshape(B, NO, N, D)
    repT = repT_f.reshape(B, NO, N)
    return _stage_e(q, fp, nlinT, repT, kg, pepack, sel, selT, Wo, bo, gamma, beta)
